# baseline (speedup 1.0000x reference)
import numpy as np
import jax
import jax.numpy as jnp
from jax.experimental import pallas as pl
from jax.experimental.pallas import tpu as pltpu

TB = 1024
BF16 = jnp.bfloat16
F32 = jnp.float32


def _onehot(shape, cond):
    m = np.zeros(shape, np.float32)
    it = np.nditer(m, flags=["multi_index"])
    for _ in it:
        if cond(*it.multi_index):
            m[it.multi_index] = 1.0
    return m


_A1 = _onehot((8, 2, 2, 5), lambda d, t, r, i: d == 2 * t + r + i)
_B1 = _onehot((32, 2, 14, 5), lambda u, p, px, j: u == 2 * px + p + j)
_A2 = _onehot((6, 2, 5), lambda d, r, i: d == r + i)
_B2 = _onehot((14, 2, 5, 5), lambda u, p, px, j: u == 2 * px + p + j)


def _bake_conv1(w1s):
    w1t = w1s.reshape(6, 5, 5).transpose(1, 2, 0)
    u = (_A1.reshape(32, 5) @ w1t.reshape(5, 30)).reshape(8, 2, 2, 5, 6)
    u = u.transpose(3, 0, 1, 2, 4).reshape(5, 192)
    v = (_B1.reshape(896, 5) @ u).reshape(32, 2, 14, 8, 2, 2, 6)
    v = v.transpose(3, 0, 4, 5, 1, 6, 2).reshape(256, 8, 84)
    return jnp.pad(v, ((0, 0), (0, 0), (0, 44))).reshape(256, 1024)


def _bake_conv2(w2s):
    w2t = w2s.reshape(16, 6, 5, 5).transpose(2, 3, 1, 0)
    u = (_A2.reshape(12, 5) @ w2t.reshape(5, 480)).reshape(6, 2, 5, 6, 16)
    u = u.transpose(2, 0, 1, 3, 4).reshape(5, 1152)
    v = (_B2.reshape(140, 5) @ u).reshape(14, 2, 5, 6, 2, 6, 16)
    v = v.transpose(3, 5, 0, 4, 1, 6, 2).reshape(504, 4, 80)
    return jnp.pad(v, ((0, 0), (0, 0), (0, 48))).reshape(504, 512)


def _bake_fc1(wf1):
    t = jnp.pad(wf1.reshape(128, 16, 72), ((0, 0), (0, 0), (0, 8)))
    t = t.reshape(128, 16, 5, 16)[:, :, :, :5]
    return t.transpose(0, 2, 1, 3).reshape(128, 400).T


def _lenet_mxu_kernel(xw_ref, w1g_ref, w2p_ref, wf1_ref, wf2_ref, wf3_ref,
                      bias_ref, o_ref, xq_ref, a1_ref, a2_ref):
    b1 = bias_ref[0:1, 0:84]
    b2 = bias_ref[0:1, 84:164]
    bf1 = bias_ref[0:1, 164:292]
    bf2 = bias_ref[0:1, 292:420]
    bf3 = bias_ref[0:1, 420:430]

    xq_ref[:, 0:66] = jnp.zeros((xq_ref.shape[0], 66), BF16)
    xq_ref[:, 962:1024] = jnp.zeros((xq_ref.shape[0], 62), BF16)
    xq_ref[:, 66:962] = xw_ref[...]

    for g in range(7):
        lhs = xq_ref[:, 128 * g:128 * g + 256]
        y = jnp.dot(lhs, w1g_ref[...], preferred_element_type=F32)
        for t in range(2):
            s = 512 * t
            m = jnp.maximum(
                jnp.maximum(y[:, s:s + 84], y[:, s + 128:s + 212]),
                jnp.maximum(y[:, s + 256:s + 340], y[:, s + 384:s + 468]))
            a = jnp.maximum(m + b1, 0.0).astype(BF16)
            py = 2 * g + t
            a1_ref[:, 84 * py:84 * py + 84] = a

    for py in range(5):
        lhs = a1_ref[:, 168 * py:168 * py + 504]
        y = jnp.dot(lhs, w2p_ref[...], preferred_element_type=F32)
        m = jnp.maximum(jnp.maximum(y[:, 0:80], y[:, 128:208]),
                        jnp.maximum(y[:, 256:336], y[:, 384:464]))
        a = jnp.maximum(m + b2, 0.0).astype(BF16)
        a2_ref[:, 80 * py:80 * py + 80] = a

    h = jnp.dot(a2_ref[...], wf1_ref[...], preferred_element_type=F32)
    h = jnp.maximum(h + bf1, 0.0).astype(BF16)
    h = jnp.dot(h, wf2_ref[...], preferred_element_type=F32)
    h = jnp.maximum(h + bf2, 0.0).astype(BF16)
    o_ref[...] = (jnp.dot(h, wf3_ref[...], preferred_element_type=F32)
                  + bf3)


def kernel(x, w1s, b1s, w2s, b2s, wf1, bf1, wf2, bf2, wf3, bf3):
    B = x.shape[0]
    nb = -(-B // TB)
    bp = nb * TB

    xb = x[:, 0].astype(BF16)
    if bp != B:
        xb = jnp.pad(xb, ((0, bp - B), (0, 0), (0, 0)))
    xp = jnp.concatenate(
        [xb, jnp.zeros((bp, 28, 4), BF16)], axis=2).reshape(bp, 896)

    w1g = _bake_conv1(w1s).astype(BF16)
    w2p = _bake_conv2(w2s).astype(BF16)
    wf1m = _bake_fc1(wf1).astype(BF16)
    wf2t = wf2.T.astype(BF16)
    wf3t = wf3.T[:, :10].astype(BF16)
    ball = jnp.concatenate([
        jnp.repeat(b1s, 14), jnp.repeat(b2s, 5),
        bf1[:, 0], bf2[:, 0], bf3[:10, 0]])
    ballr = jnp.broadcast_to(ball[None, :], (8, 430))

    flops = bp * 2 * (28 * 28 * 25 * 6 + 10 * 10 * 150 * 16
                      + 400 * 120 + 120 * 84 + 84 * 10)
    bytes_accessed = 2 * bp * 896 + 4 * bp * 10 + 2 * (
        256 * 1024 + 504 * 512 + 400 * 128 + 128 * 128 + 128 * 10)

    full = pl.BlockSpec(index_map=lambda b: (0, 0))
    out = pl.pallas_call(
        _lenet_mxu_kernel,
        out_shape=jax.ShapeDtypeStruct((bp, 10), F32),
        grid=(nb,),
        in_specs=[
            pl.BlockSpec((TB, 896), lambda b: (b, 0)),
            full, full, full, full, full, full,
        ],
        out_specs=pl.BlockSpec((TB, 10), lambda b: (b, 0)),
        scratch_shapes=[
            pltpu.VMEM((TB, 1024), BF16),
            pltpu.VMEM((TB, 1176), BF16),
            pltpu.VMEM((TB, 400), BF16),
        ],
        compiler_params=pltpu.CompilerParams(
            dimension_semantics=("parallel",),
            vmem_limit_bytes=48 * 1024 * 1024),
        cost_estimate=pl.CostEstimate(
            flops=flops, transcendentals=0, bytes_accessed=bytes_accessed),
    )(xp, w1g, w2p, wf1m, wf2t, wf3t, ballr)

    return out if bp == B else out[:B]


# speedup vs baseline: 1.0044x; 1.0044x over previous
import numpy as np
import jax
import jax.numpy as jnp
from jax.experimental import pallas as pl
from jax.experimental.pallas import tpu as pltpu

TB = 2048
BF16 = jnp.bfloat16
F32 = jnp.float32


def _onehot(shape, cond):
    m = np.zeros(shape, np.float32)
    it = np.nditer(m, flags=["multi_index"])
    for _ in it:
        if cond(*it.multi_index):
            m[it.multi_index] = 1.0
    return m


_A1 = _onehot((8, 2, 2, 5), lambda d, t, r, i: d == 2 * t + r + i)
_B1 = _onehot((32, 2, 14, 5), lambda u, p, px, j: u == 2 * px + p + j)
_A2 = _onehot((6, 2, 5), lambda d, r, i: d == r + i)
_B2 = _onehot((14, 2, 5, 5), lambda u, p, px, j: u == 2 * px + p + j)


def _bake_conv1(w1s):
    w1t = w1s.reshape(6, 5, 5).transpose(1, 2, 0)
    u = (_A1.reshape(32, 5) @ w1t.reshape(5, 30)).reshape(8, 2, 2, 5, 6)
    u = u.transpose(3, 0, 1, 2, 4).reshape(5, 192)
    v = (_B1.reshape(896, 5) @ u).reshape(32, 2, 14, 8, 2, 2, 6)
    v = v.transpose(3, 0, 4, 5, 1, 6, 2).reshape(256, 8, 84)
    return jnp.pad(v, ((0, 0), (0, 0), (0, 44))).reshape(256, 1024)


def _bake_conv2(w2s):
    w2t = w2s.reshape(16, 6, 5, 5).transpose(2, 3, 1, 0)
    u = (_A2.reshape(12, 5) @ w2t.reshape(5, 480)).reshape(6, 2, 5, 6, 16)
    u = u.transpose(2, 0, 1, 3, 4).reshape(5, 1152)
    v = (_B2.reshape(140, 5) @ u).reshape(14, 2, 5, 6, 2, 6, 16)
    v = v.transpose(3, 5, 0, 4, 1, 6, 2).reshape(504, 4, 80)
    return jnp.pad(v, ((0, 0), (0, 0), (0, 48))).reshape(504, 512)


def _bake_fc1(wf1):
    t = jnp.pad(wf1.reshape(128, 16, 72), ((0, 0), (0, 0), (0, 8)))
    t = t.reshape(128, 16, 5, 16)[:, :, :, :5]
    return t.transpose(0, 2, 1, 3).reshape(128, 400).T


def _lenet_mxu_kernel(xw_ref, w1g_ref, w2p_ref, wf1_ref, wf2_ref, wf3_ref,
                      bias_ref, o_ref, xq_ref, a1_ref, a2_ref):
    b1 = bias_ref[0:1, 0:84]
    b2 = bias_ref[0:1, 84:164]
    bf1 = bias_ref[0:1, 164:292]
    bf2 = bias_ref[0:1, 292:420]
    bf3 = bias_ref[0:1, 420:430]

    xq_ref[:, 0:66] = jnp.zeros((xq_ref.shape[0], 66), BF16)
    xq_ref[:, 962:1024] = jnp.zeros((xq_ref.shape[0], 62), BF16)
    xq_ref[:, 66:962] = xw_ref[...]

    for g in range(7):
        lhs = xq_ref[:, 128 * g:128 * g + 256]
        y = jnp.dot(lhs, w1g_ref[...], preferred_element_type=F32)
        for t in range(2):
            s = 512 * t
            m = jnp.maximum(
                jnp.maximum(y[:, s:s + 84], y[:, s + 128:s + 212]),
                jnp.maximum(y[:, s + 256:s + 340], y[:, s + 384:s + 468]))
            a = jnp.maximum(m + b1, 0.0).astype(BF16)
            py = 2 * g + t
            a1_ref[:, 84 * py:84 * py + 84] = a

    for py in range(5):
        lhs = a1_ref[:, 168 * py:168 * py + 504]
        y = jnp.dot(lhs, w2p_ref[...], preferred_element_type=F32)
        m = jnp.maximum(jnp.maximum(y[:, 0:80], y[:, 128:208]),
                        jnp.maximum(y[:, 256:336], y[:, 384:464]))
        a = jnp.maximum(m + b2, 0.0).astype(BF16)
        a2_ref[:, 80 * py:80 * py + 80] = a

    h = jnp.dot(a2_ref[...], wf1_ref[...], preferred_element_type=F32)
    h = jnp.maximum(h + bf1, 0.0).astype(BF16)
    h = jnp.dot(h, wf2_ref[...], preferred_element_type=F32)
    h = jnp.maximum(h + bf2, 0.0).astype(BF16)
    o_ref[...] = (jnp.dot(h, wf3_ref[...], preferred_element_type=F32)
                  + bf3)


def kernel(x, w1s, b1s, w2s, b2s, wf1, bf1, wf2, bf2, wf3, bf3):
    B = x.shape[0]
    nb = -(-B // TB)
    bp = nb * TB

    xb = x[:, 0].astype(BF16)
    if bp != B:
        xb = jnp.pad(xb, ((0, bp - B), (0, 0), (0, 0)))
    xp = jnp.concatenate(
        [xb, jnp.zeros((bp, 28, 4), BF16)], axis=2).reshape(bp, 896)

    w1g = _bake_conv1(w1s).astype(BF16)
    w2p = _bake_conv2(w2s).astype(BF16)
    wf1m = _bake_fc1(wf1).astype(BF16)
    wf2t = wf2.T.astype(BF16)
    wf3t = wf3.T[:, :10].astype(BF16)
    ball = jnp.concatenate([
        jnp.repeat(b1s, 14), jnp.repeat(b2s, 5),
        bf1[:, 0], bf2[:, 0], bf3[:10, 0]])
    ballr = jnp.broadcast_to(ball[None, :], (8, 430))

    flops = bp * 2 * (28 * 28 * 25 * 6 + 10 * 10 * 150 * 16
                      + 400 * 120 + 120 * 84 + 84 * 10)
    bytes_accessed = 2 * bp * 896 + 4 * bp * 10 + 2 * (
        256 * 1024 + 504 * 512 + 400 * 128 + 128 * 128 + 128 * 10)

    full = pl.BlockSpec(index_map=lambda b: (0, 0))
    out = pl.pallas_call(
        _lenet_mxu_kernel,
        out_shape=jax.ShapeDtypeStruct((bp, 10), F32),
        grid=(nb,),
        in_specs=[
            pl.BlockSpec((TB, 896), lambda b: (b, 0)),
            full, full, full, full, full, full,
        ],
        out_specs=pl.BlockSpec((TB, 10), lambda b: (b, 0)),
        scratch_shapes=[
            pltpu.VMEM((TB, 1024), BF16),
            pltpu.VMEM((TB, 1176), BF16),
            pltpu.VMEM((TB, 400), BF16),
        ],
        compiler_params=pltpu.CompilerParams(
            dimension_semantics=("parallel",),
            vmem_limit_bytes=48 * 1024 * 1024),
        cost_estimate=pl.CostEstimate(
            flops=flops, transcendentals=0, bytes_accessed=bytes_accessed),
    )(xp, w1g, w2p, wf1m, wf2t, wf3t, ballr)

    return out if bp == B else out[:B]


# speedup vs baseline: 1.0330x; 1.0285x over previous
import numpy as np
import jax
import jax.numpy as jnp
from jax.experimental import pallas as pl
from jax.experimental.pallas import tpu as pltpu

TB = 2048
BF16 = jnp.bfloat16
F32 = jnp.float32


def _onehot(shape, cond):
    m = np.zeros(shape, np.float32)
    it = np.nditer(m, flags=["multi_index"])
    for _ in it:
        if cond(*it.multi_index):
            m[it.multi_index] = 1.0
    return m


_A1 = _onehot((8, 2, 2, 5), lambda d, t, r, i: d == 2 * t + r + i)
_B1 = _onehot((32, 2, 14, 5), lambda u, p, px, j: u == 2 * px + p + j)
_A2 = _onehot((6, 2, 5), lambda d, r, i: d == r + i)
_B2 = _onehot((14, 2, 5, 5), lambda u, p, px, j: u == 2 * px + p + j)


def _bake_conv1(w1s):
    v = jnp.einsum('dtri,upxj,cij->dutrpcx', _A1, _B1,
                   w1s.reshape(6, 5, 5)).reshape(256, 8, 84)
    return jnp.pad(v, ((0, 0), (0, 0), (0, 44))).reshape(256, 1024)


def _bake_conv2(w2s):
    v = jnp.einsum('dri,upxj,ocij->dcurpox', _A2, _B2,
                   w2s.reshape(16, 6, 5, 5)).reshape(504, 4, 80)
    return jnp.pad(v, ((0, 0), (0, 0), (0, 48))).reshape(504, 512)


def _bake_fc1(wf1):
    t = jnp.pad(wf1.reshape(128, 16, 72), ((0, 0), (0, 0), (0, 8)))
    t = t.reshape(128, 16, 5, 16)[:, :, :, :5]
    return t.transpose(0, 2, 1, 3).reshape(128, 400).T


def _lenet_mxu_kernel(xw_ref, w1g_ref, w2p_ref, wf1_ref, wf2_ref, wf3_ref,
                      bias_ref, o_ref, xq_ref, a1_ref, a2_ref):
    b1 = bias_ref[0:1, 0:84]
    b2 = bias_ref[0:1, 84:164]
    bf1 = bias_ref[0:1, 164:292]
    bf2 = bias_ref[0:1, 292:420]
    bf3 = bias_ref[0:1, 420:430]

    xq_ref[:, 0:66] = jnp.zeros((xq_ref.shape[0], 66), BF16)
    xq_ref[:, 962:1024] = jnp.zeros((xq_ref.shape[0], 62), BF16)
    xq_ref[:, 66:962] = xw_ref[...]

    for g in range(7):
        lhs = xq_ref[:, 128 * g:128 * g + 256]
        y = jnp.dot(lhs, w1g_ref[...], preferred_element_type=F32)
        for t in range(2):
            s = 512 * t
            m = jnp.maximum(
                jnp.maximum(y[:, s:s + 84], y[:, s + 128:s + 212]),
                jnp.maximum(y[:, s + 256:s + 340], y[:, s + 384:s + 468]))
            a = jnp.maximum(m + b1, 0.0).astype(BF16)
            py = 2 * g + t
            a1_ref[:, 84 * py:84 * py + 84] = a

    for py in range(5):
        lhs = a1_ref[:, 168 * py:168 * py + 504]
        y = jnp.dot(lhs, w2p_ref[...], preferred_element_type=F32)
        m = jnp.maximum(jnp.maximum(y[:, 0:80], y[:, 128:208]),
                        jnp.maximum(y[:, 256:336], y[:, 384:464]))
        a = jnp.maximum(m + b2, 0.0).astype(BF16)
        a2_ref[:, 80 * py:80 * py + 80] = a

    h = jnp.dot(a2_ref[...], wf1_ref[...], preferred_element_type=F32)
    h = jnp.maximum(h + bf1, 0.0).astype(BF16)
    h = jnp.dot(h, wf2_ref[...], preferred_element_type=F32)
    h = jnp.maximum(h + bf2, 0.0).astype(BF16)
    o_ref[...] = (jnp.dot(h, wf3_ref[...], preferred_element_type=F32)
                  + bf3)


def kernel(x, w1s, b1s, w2s, b2s, wf1, bf1, wf2, bf2, wf3, bf3):
    B = x.shape[0]
    nb = -(-B // TB)
    bp = nb * TB

    xb = x[:, 0].astype(BF16)
    if bp != B:
        xb = jnp.pad(xb, ((0, bp - B), (0, 0), (0, 0)))
    xp = jnp.concatenate(
        [xb, jnp.zeros((bp, 28, 4), BF16)], axis=2).reshape(bp, 896)

    w1g = _bake_conv1(w1s).astype(BF16)
    w2p = _bake_conv2(w2s).astype(BF16)
    wf1m = _bake_fc1(wf1).astype(BF16)
    wf2t = wf2.T.astype(BF16)
    wf3t = wf3.T[:, :10].astype(BF16)
    ball = jnp.concatenate([
        jnp.repeat(b1s, 14), jnp.repeat(b2s, 5),
        bf1[:, 0], bf2[:, 0], bf3[:10, 0]])
    ballr = jnp.broadcast_to(ball[None, :], (8, 430))

    flops = bp * 2 * (28 * 28 * 25 * 6 + 10 * 10 * 150 * 16
                      + 400 * 120 + 120 * 84 + 84 * 10)
    bytes_accessed = 2 * bp * 896 + 4 * bp * 10 + 2 * (
        256 * 1024 + 504 * 512 + 400 * 128 + 128 * 128 + 128 * 10)

    full = pl.BlockSpec(index_map=lambda b: (0, 0))
    out = pl.pallas_call(
        _lenet_mxu_kernel,
        out_shape=jax.ShapeDtypeStruct((bp, 10), F32),
        grid=(nb,),
        in_specs=[
            pl.BlockSpec((TB, 896), lambda b: (b, 0)),
            full, full, full, full, full, full,
        ],
        out_specs=pl.BlockSpec((TB, 10), lambda b: (b, 0)),
        scratch_shapes=[
            pltpu.VMEM((TB, 1024), BF16),
            pltpu.VMEM((TB, 1176), BF16),
            pltpu.VMEM((TB, 400), BF16),
        ],
        compiler_params=pltpu.CompilerParams(
            dimension_semantics=("parallel",),
            vmem_limit_bytes=48 * 1024 * 1024),
        cost_estimate=pl.CostEstimate(
            flops=flops, transcendentals=0, bytes_accessed=bytes_accessed),
    )(xp, w1g, w2p, wf1m, wf2t, wf3t, ballr)

    return out if bp == B else out[:B]
